# baseline (speedup 1.0000x reference)
import jax
import jax.numpy as jnp
from jax import lax
from jax.experimental import pallas as pl
from jax.experimental.pallas import tpu as pltpu
from jax.experimental.pallas import tpu_sc as plsc


_CHUNK_ROWS = 32
_NSLOTS = 3
_SC_ROWS = 3072


def _sc_body(table_hbm, out_hbm, buf_v, sem_in, sem_out):
    info = plsc.get_sparse_core_info()
    nw = info.num_cores * info.num_subcores
    wid = lax.axis_index("s") * info.num_cores + lax.axis_index("c")
    sc_base = table_hbm.shape[0] - _SC_ROWS
    rows = _SC_ROWS // nw
    nchunk = rows // _CHUNK_ROWS
    base = sc_base + wid * rows

    def cin(j):
        return pltpu.make_async_copy(
            table_hbm.at[pl.ds(base + j * _CHUNK_ROWS, _CHUNK_ROWS)],
            buf_v.at[j % _NSLOTS], sem_in)

    def cout(j):
        return pltpu.make_async_copy(
            buf_v.at[j % _NSLOTS],
            out_hbm.at[pl.ds(wid * rows + j * _CHUNK_ROWS, _CHUNK_ROWS)],
            sem_out)

    outs = []
    for j in range(min(_NSLOTS, nchunk)):
        cin(j).start()
    for j in range(nchunk):
        cin(j).wait()
        c = cout(j)
        c.start()
        outs.append(c)
        k = j + 1
        if _NSLOTS <= k < nchunk:
            outs[k - _NSLOTS].wait()
            cin(k).start()
    for c in outs[-_NSLOTS:]:
        c.wait()


def _tc_body(t_ref, o_ref):
    o_ref[...] = t_ref[...]


def kernel(inputs, table):
    rows, hid = table.shape
    tc_rows = rows - _SC_ROWS
    blk = 256
    mesh = plsc.VectorSubcoreMesh(core_axis_name="c", subcore_axis_name="s")
    sc_part = pl.kernel(
        _sc_body,
        out_type=jax.ShapeDtypeStruct((_SC_ROWS, hid), table.dtype),
        scratch_types=[
            pltpu.VMEM((_NSLOTS, _CHUNK_ROWS, hid), table.dtype),
            pltpu.SemaphoreType.DMA,
            pltpu.SemaphoreType.DMA,
        ],
        mesh=mesh,
    )(table)
    tc_part = pl.pallas_call(
        _tc_body,
        grid=(tc_rows // blk,),
        in_specs=[pl.BlockSpec((blk, hid), lambda i: (i, 0))],
        out_specs=pl.BlockSpec((blk, hid), lambda i: (i, 0)),
        out_shape=jax.ShapeDtypeStruct((tc_rows, hid), table.dtype),
    )(table)
    return jnp.concatenate([tc_part, sc_part], axis=0)[None]


# speedup vs baseline: 1.5398x; 1.5398x over previous
import jax
import jax.numpy as jnp
from jax import lax
from jax.experimental import pallas as pl
from jax.experimental.pallas import tpu as pltpu
from jax.experimental.pallas import tpu_sc as plsc


_CHUNK_ROWS = 32
_NSLOTS = 3


def _copy_body(table_hbm, out_hbm, buf_v, sem_in, sem_out):
    info = plsc.get_sparse_core_info()
    nw = info.num_cores * info.num_subcores
    wid = lax.axis_index("s") * info.num_cores + lax.axis_index("c")
    rows = table_hbm.shape[0] // nw
    nchunk = rows // _CHUNK_ROWS
    base = wid * rows

    def cin(j):
        return pltpu.make_async_copy(
            table_hbm.at[pl.ds(base + j * _CHUNK_ROWS, _CHUNK_ROWS)],
            buf_v.at[j % _NSLOTS], sem_in)

    def cout(j):
        return pltpu.make_async_copy(
            buf_v.at[j % _NSLOTS],
            out_hbm.at[pl.ds(base + j * _CHUNK_ROWS, _CHUNK_ROWS)], sem_out)

    outs = []
    for j in range(min(_NSLOTS, nchunk)):
        cin(j).start()
    for j in range(nchunk):
        cin(j).wait()
        c = cout(j)
        c.start()
        outs.append(c)
        k = j + 1
        if _NSLOTS <= k < nchunk:
            outs[k - _NSLOTS].wait()
            cin(k).start()
    for c in outs[-_NSLOTS:]:
        c.wait()


def kernel(inputs, table):
    seq_len = inputs.shape[1]
    assert seq_len == table.shape[0]
    mesh = plsc.VectorSubcoreMesh(core_axis_name="c", subcore_axis_name="s")
    out = pl.kernel(
        _copy_body,
        out_type=jax.ShapeDtypeStruct(table.shape, table.dtype),
        scratch_types=[
            pltpu.VMEM((_NSLOTS, _CHUNK_ROWS, table.shape[1]), table.dtype),
            pltpu.SemaphoreType.DMA,
            pltpu.SemaphoreType.DMA,
        ],
        mesh=mesh,
    )(table)
    return out[None]
